# baseline (speedup 1.0000x reference)
import dataclasses
import functools

import jax
import jax.numpy as jnp
from jax import lax
from jax.experimental import pallas as pl
from jax.experimental.pallas import tpu as pltpu
from jax.experimental.pallas import tpu_sc as plsc

F = 26
D = 16
B = 4096
TD = F * D
NC = 2
NS = 16
NW = NC * NS
EPW = B // NW
CHUNK = 4
RPC = CHUNK * F
NCHUNK = EPW // CHUNK
GROUPS = EPW // D


_cp = pltpu.CompilerParams()
if "needs_layout_passes" in pltpu.CompilerParams.__dataclass_fields__:
    _cp = dataclasses.replace(_cp, needs_layout_passes=False)
if "use_tc_tiling_on_sc" in pltpu.CompilerParams.__dataclass_fields__:
    _cp = dataclasses.replace(_cp, use_tc_tiling_on_sc=False)


@functools.partial(
    pl.kernel,
    out_type=jax.ShapeDtypeStruct((B // D, D), jnp.float32),
    mesh=plsc.VectorSubcoreMesh(core_axis_name="c", subcore_axis_name="s"),
    compiler_params=_cp,
    scratch_types=[
        pltpu.VMEM((EPW * F,), jnp.int32),
        pltpu.VMEM((2, RPC, F, D), jnp.float32),
        pltpu.VMEM((2, RPC, D), jnp.float32),
        pltpu.VMEM((D, EPW), jnp.float32),
        pltpu.VMEM((GROUPS, D), jnp.float32),
        pltpu.SemaphoreType.DMA((2,)),
        pltpu.SemaphoreType.DMA((2,)),
    ],
)
def _ffm_sc(tab_hbm, w_hbm, idx_hbm, out_hbm,
            idx_v, rows, wrows, acc_t, out_v, rsem, wsem):
    wid = lax.axis_index("s") * NC + lax.axis_index("c")
    pltpu.sync_copy(idx_hbm.at[pl.ds(wid * (EPW * F), EPW * F)], idx_v)

    def copies(c, b):
        off = c * RPC if isinstance(c, int) else pl.multiple_of(c * RPC, 8)
        isl = idx_v.at[pl.ds(off, RPC)]
        return (
            pltpu.make_async_copy(tab_hbm.at[isl], rows.at[b], rsem.at[b]),
            pltpu.make_async_copy(w_hbm.at[isl], wrows.at[b], wsem.at[b]),
        )

    for cp in copies(0, 0):
        cp.start()
    for cp in copies(1, 1):
        cp.start()

    @pl.loop(0, NCHUNK)
    def _chunk(c):
        b = lax.rem(c, 2)
        for cp in copies(c, b):
            cp.wait()

        @pl.loop(0, CHUNK)
        def _elem(e):
            base = e * F
            accs = [jnp.zeros((D,), jnp.float32) for _ in range(8)]
            k = 0
            for i in range(F):
                for j in range(i + 1, F):
                    accs[k % 8] = accs[k % 8] + (rows[b, base + i, j, :]
                                                 * rows[b, base + j, i, :])
                    k += 1
            for i in range(F):
                accs[i % 8] = accs[i % 8] + wrows[b, base + i, :]
            acc = ((accs[0] + accs[1]) + (accs[2] + accs[3])) + (
                (accs[4] + accs[5]) + (accs[6] + accs[7]))
            n = c * CHUNK + e
            plsc.store_scatter(
                acc_t,
                [lax.iota(jnp.int32, D), jnp.full((D,), n, jnp.int32)],
                acc,
            )

        @pl.when(c + 2 < NCHUNK)
        def _refill():
            for cp in copies(c + 2, b):
                cp.start()

    for g in range(GROUPS):
        r = jnp.zeros((D,), jnp.float32)
        for dd in range(D):
            r = r + acc_t[dd, pl.ds(g * D, D)]
        out_v[g, :] = r
    pltpu.sync_copy(out_v, out_hbm.at[pl.ds(wid * GROUPS, GROUPS)])


_cpt = dataclasses.replace(_cp, use_tc_tiling_on_sc=True)

RT = 26000
RB = 128
RBTD = RB * TD
NBLK = RT // RB
TAIL = RT - NBLK * RB
NITER = (NBLK + NW - 1) // NW


HB = RB // 2
PS = HB + 1
HBTD = HB * TD


@functools.partial(
    pl.kernel,
    out_type=jax.ShapeDtypeStruct((RT * TD,), jnp.float32),
    mesh=plsc.VectorSubcoreMesh(core_axis_name="c", subcore_axis_name="s"),
    compiler_params=_cpt,
    scratch_types=[
        pltpu.VMEM((F, D, RB), jnp.float32),
        pltpu.VMEM((F * D * PS,), jnp.float32),
        pltpu.VMEM((HBTD,), jnp.float32),
    ],
)
def _transpose_sc(tv_hbm, tail_hbm, out_hbm, tin, tpad, tout):
    wid = lax.axis_index("s") * NC + lax.axis_index("c")
    lanes = lax.iota(jnp.int32, D)

    @pl.loop(0, NITER)
    def _turn(t):
        blk = wid + t * NW

        @pl.when(blk < NBLK)
        def _():
            roff = pl.multiple_of(blk * RB, RB)
            pltpu.sync_copy(tv_hbm.at[:, :, pl.ds(roff, RB)], tin)

            @pl.loop(0, 2)
            def _half(h):
                @pl.loop(0, HB, step=D)
                def _ra(rsub):
                    src = h * HB + rsub
                    idx0 = (rsub + lanes).astype(jnp.int32)
                    for f in range(F):
                        vs = [tin[f, dd, pl.ds(src, D)] for dd in range(D)]
                        for dd in range(D):
                            plsc.store_scatter(
                                tpad, [idx0 + (f * D + dd) * PS], vs[dd])

                @pl.loop(0, HB)
                def _rb(r):
                    for fb in range(0, F, 13):
                        gs = [
                            plsc.load_gather(
                                tpad,
                                [((f * D + lanes) * PS + r).astype(jnp.int32)])
                            for f in range(fb, fb + 13)
                        ]
                        for k, f in enumerate(range(fb, fb + 13)):
                            tout[pl.ds(r * TD + f * D, D)] = gs[k]

                pltpu.sync_copy(
                    tout,
                    out_hbm.at[pl.ds(blk * RBTD + h * HBTD, HBTD)])

    @pl.when(wid == NW - 1)
    def _tail():
        pltpu.sync_copy(tail_hbm, tout.at[pl.ds(0, TAIL * TD)])
        pltpu.sync_copy(tout.at[pl.ds(0, TAIL * TD)],
                        out_hbm.at[pl.ds(NBLK * RBTD, TAIL * TD)])


def kernel(inputs, embedding_table, linear_w, linear_b):
    rows_total, nf, d = embedding_table.shape
    offsets = (rows_total // nf) * jnp.arange(nf, dtype=jnp.int32)
    gidx = (inputs + offsets[None, :]).reshape(-1)
    w_pad = jnp.pad(linear_w, ((0, 0), (0, d - 1)))
    tv = jnp.transpose(embedding_table, (1, 2, 0))
    tail = embedding_table[NBLK * RB:].reshape(-1)
    tbl = _transpose_sc(tv, tail).reshape(rows_total, nf, d)
    out = _ffm_sc(tbl, w_pad, gidx)
    return out.reshape(B, 1) + linear_b


# speedup vs baseline: 1.8841x; 1.8841x over previous
import dataclasses
import functools

import jax
import jax.numpy as jnp
from jax import lax
from jax.experimental import pallas as pl
from jax.experimental.pallas import tpu as pltpu
from jax.experimental.pallas import tpu_sc as plsc

F = 26
D = 16
B = 4096
TD = F * D
NC = 2
NS = 16
NW = NC * NS
EPW = B // NW
CHUNK = 4
RPC = CHUNK * F
NCHUNK = EPW // CHUNK
GROUPS = EPW // D


_cp = pltpu.CompilerParams()
if "needs_layout_passes" in pltpu.CompilerParams.__dataclass_fields__:
    _cp = dataclasses.replace(_cp, needs_layout_passes=False)
if "use_tc_tiling_on_sc" in pltpu.CompilerParams.__dataclass_fields__:
    _cp = dataclasses.replace(_cp, use_tc_tiling_on_sc=False)


@functools.partial(
    pl.kernel,
    out_type=jax.ShapeDtypeStruct((B // D, D), jnp.float32),
    mesh=plsc.VectorSubcoreMesh(core_axis_name="c", subcore_axis_name="s"),
    compiler_params=_cp,
    scratch_types=[
        pltpu.VMEM((EPW * F,), jnp.int32),
        pltpu.VMEM((2, RPC, F, D), jnp.float32),
        pltpu.VMEM((2, RPC, D), jnp.float32),
        pltpu.VMEM((D, EPW), jnp.float32),
        pltpu.VMEM((GROUPS, D), jnp.float32),
        pltpu.SemaphoreType.DMA((2,)),
        pltpu.SemaphoreType.DMA((2,)),
    ],
)
def _ffm_sc(tab_hbm, w_hbm, idx_hbm, out_hbm,
            idx_v, rows, wrows, acc_t, out_v, rsem, wsem):
    wid = lax.axis_index("s") * NC + lax.axis_index("c")
    pltpu.sync_copy(idx_hbm.at[pl.ds(wid * (EPW * F), EPW * F)], idx_v)

    def copies(c, b):
        off = c * RPC if isinstance(c, int) else pl.multiple_of(c * RPC, 8)
        isl = idx_v.at[pl.ds(off, RPC)]
        return (
            pltpu.make_async_copy(tab_hbm.at[isl], rows.at[b], rsem.at[b]),
            pltpu.make_async_copy(w_hbm.at[isl], wrows.at[b], wsem.at[b]),
        )

    for cp in copies(0, 0):
        cp.start()
    for cp in copies(1, 1):
        cp.start()

    @pl.loop(0, NCHUNK)
    def _chunk(c):
        b = lax.rem(c, 2)
        for cp in copies(c, b):
            cp.wait()

        @pl.loop(0, CHUNK)
        def _elem(e):
            base = e * F
            acc = jnp.zeros((D,), jnp.float32)
            for i in range(F):
                for j in range(i + 1, F):
                    acc = acc + (rows[b, base + i, j, :]
                                 * rows[b, base + j, i, :])
            for i in range(F):
                acc = acc + wrows[b, base + i, :]
            n = c * CHUNK + e
            plsc.store_scatter(
                acc_t,
                [lax.iota(jnp.int32, D), jnp.full((D,), n, jnp.int32)],
                acc,
            )

        @pl.when(c + 2 < NCHUNK)
        def _refill():
            for cp in copies(c + 2, b):
                cp.start()

    for g in range(GROUPS):
        r = jnp.zeros((D,), jnp.float32)
        for dd in range(D):
            r = r + acc_t[dd, pl.ds(g * D, D)]
        out_v[g, :] = r
    pltpu.sync_copy(out_v, out_hbm.at[pl.ds(wid * GROUPS, GROUPS)])


_cpt = dataclasses.replace(_cp, use_tc_tiling_on_sc=True)

RT = 26000
RB = 128
RBTD = RB * TD
NBLK = RT // RB
TAIL = RT - NBLK * RB
NITER = (NBLK + NW - 1) // NW


HB = RB // 2
PS = HB + 1
HBTD = HB * TD


@functools.partial(
    pl.kernel,
    out_type=jax.ShapeDtypeStruct((RT * TD,), jnp.float32),
    mesh=plsc.VectorSubcoreMesh(core_axis_name="c", subcore_axis_name="s"),
    compiler_params=_cpt,
    scratch_types=[
        pltpu.VMEM((F, D, RB), jnp.float32),
        pltpu.VMEM((F * D * PS,), jnp.float32),
        pltpu.VMEM((HBTD,), jnp.float32),
    ],
)
def _transpose_sc(tv_hbm, tail_hbm, out_hbm, tin, tpad, tout):
    wid = lax.axis_index("s") * NC + lax.axis_index("c")
    lanes = lax.iota(jnp.int32, D)

    @pl.loop(0, NITER)
    def _turn(t):
        blk = wid + t * NW

        @pl.when(blk < NBLK)
        def _():
            roff = pl.multiple_of(blk * RB, RB)
            pltpu.sync_copy(tv_hbm.at[:, :, pl.ds(roff, RB)], tin)

            @pl.loop(0, 2)
            def _half(h):
                @pl.loop(0, HB, step=D)
                def _ra(rsub):
                    src = h * HB + rsub
                    idx0 = (rsub + lanes).astype(jnp.int32)
                    for f in range(F):
                        vs = [tin[f, dd, pl.ds(src, D)] for dd in range(D)]
                        for dd in range(D):
                            plsc.store_scatter(
                                tpad, [idx0 + (f * D + dd) * PS], vs[dd])

                @pl.loop(0, HB)
                def _rb(r):
                    for fb in range(0, F, 13):
                        gs = [
                            plsc.load_gather(
                                tpad,
                                [((f * D + lanes) * PS + r).astype(jnp.int32)])
                            for f in range(fb, fb + 13)
                        ]
                        for k, f in enumerate(range(fb, fb + 13)):
                            tout[pl.ds(r * TD + f * D, D)] = gs[k]

                pltpu.sync_copy(
                    tout,
                    out_hbm.at[pl.ds(blk * RBTD + h * HBTD, HBTD)])

    @pl.when(wid == NW - 1)
    def _tail():
        pltpu.sync_copy(tail_hbm, tout.at[pl.ds(0, TAIL * TD)])
        pltpu.sync_copy(tout.at[pl.ds(0, TAIL * TD)],
                        out_hbm.at[pl.ds(NBLK * RBTD, TAIL * TD)])


def kernel(inputs, embedding_table, linear_w, linear_b):
    rows_total, nf, d = embedding_table.shape
    offsets = (rows_total // nf) * jnp.arange(nf, dtype=jnp.int32)
    gidx = (inputs + offsets[None, :]).reshape(-1)
    w_pad = jnp.pad(linear_w, ((0, 0), (0, d - 1)))
    tv = jnp.transpose(embedding_table, (1, 2, 0))
    tail = embedding_table[NBLK * RB:].reshape(-1)
    tbl = _transpose_sc(tv, tail).reshape(rows_total, nf, d)
    out = _ffm_sc(tbl, w_pad, gidx)
    return out.reshape(B, 1) + linear_b
